# baseline (speedup 1.0000x reference)
import functools

import jax
import jax.numpy as jnp
from jax import lax
from jax.experimental import pallas as pl
from jax.experimental.pallas import tpu as pltpu
from jax.experimental.pallas import tpu_sc as plsc

NUM_CORES = 2
NUM_SUBCORES = 16
NW = NUM_CORES * NUM_SUBCORES
BATCH = 16384
BPW = BATCH // NW
MF_DIM = 32
IDX_CHUNK = 128
NCHUNK = BPW // IDX_CHUNK
GROUPS = BPW // 16

_mesh = plsc.VectorSubcoreMesh(core_axis_name="c", subcore_axis_name="s")


@functools.partial(
    pl.kernel,
    mesh=_mesh,
    compiler_params=pltpu.CompilerParams(
        needs_layout_passes=False, use_tc_tiling_on_sc=False),
    out_type=(
        jax.ShapeDtypeStruct((NW, BPW), jnp.float32),
        jax.ShapeDtypeStruct((NW, BPW), jnp.float32),
        jax.ShapeDtypeStruct((NW, 16), jnp.float32),
    ),
    scratch_types=[
        pltpu.VMEM((NCHUNK, IDX_CHUNK), jnp.int32),
        pltpu.VMEM((NCHUNK, IDX_CHUNK), jnp.int32),
        pltpu.VMEM((BPW,), jnp.float32),
        pltpu.VMEM((BPW, MF_DIM), jnp.float32),
        pltpu.VMEM((BPW, MF_DIM), jnp.float32),
        pltpu.VMEM((BPW,), jnp.float32),
        pltpu.VMEM((BPW,), jnp.float32),
        pltpu.VMEM((16,), jnp.float32),
        pltpu.VMEM((BPW,), jnp.float32),
        pltpu.VMEM((BPW,), jnp.float32),
        pltpu.VMEM((16,), jnp.float32),
        pltpu.SemaphoreType.DMA,
    ],
)
def _pmf_sc(user_hbm, item_hbm, label_hbm, utab_hbm, itab_hbm,
            ubias_hbm, ibias_hbm, avg_hbm,
            pred_hbm, loss_hbm, obj_hbm,
            uidx_v, iidx_v, lbl_v, urows_v, irows_v, ub_v, ib_v,
            avg_v, pred_v, loss_v, obj_v, sem):
    wid = lax.axis_index("s") * NUM_CORES + lax.axis_index("c")

    pltpu.sync_copy(user_hbm.at[wid], uidx_v)
    pltpu.sync_copy(item_hbm.at[wid], iidx_v)
    pltpu.sync_copy(label_hbm.at[wid], lbl_v)
    pltpu.sync_copy(avg_hbm, avg_v)

    copies = []
    for ch in range(NCHUNK):
        dst = pl.ds(ch * IDX_CHUNK, IDX_CHUNK)
        copies.append(pltpu.make_async_copy(
            utab_hbm.at[uidx_v.at[ch]], urows_v.at[dst], sem))
        copies.append(pltpu.make_async_copy(
            itab_hbm.at[iidx_v.at[ch]], irows_v.at[dst], sem))
        copies.append(pltpu.make_async_copy(
            ubias_hbm.at[uidx_v.at[ch]], ub_v.at[dst], sem))
        copies.append(pltpu.make_async_copy(
            ibias_hbm.at[iidx_v.at[ch]], ib_v.at[dst], sem))
    for cp in copies:
        cp.start()
    for cp in copies:
        cp.wait()

    avg16 = avg_v[...]
    lane = lax.iota(jnp.int32, 16)
    cols = [jnp.full((16,), j, jnp.int32) for j in range(MF_DIM)]

    def group_body(g, obj_acc):
        rows = g * 16 + lane
        acc = jnp.zeros((16,), jnp.float32)
        for j in range(MF_DIM):
            uj = plsc.load_gather(urows_v, [rows, cols[j]])
            vj = plsc.load_gather(irows_v, [rows, cols[j]])
            acc = acc + uj * vj
        off = g * 16
        pred = acc + avg16 + ub_v[pl.ds(off, 16)] + ib_v[pl.ds(off, 16)]
        diff = pred - lbl_v[pl.ds(off, 16)]
        loss = diff * diff
        pred_v[pl.ds(off, 16)] = pred
        loss_v[pl.ds(off, 16)] = loss
        return obj_acc + loss

    obj16 = lax.fori_loop(0, GROUPS, group_body, jnp.zeros((16,), jnp.float32))
    obj_v[...] = obj16

    pltpu.sync_copy(pred_v, pred_hbm.at[wid])
    pltpu.sync_copy(loss_v, loss_hbm.at[wid])
    pltpu.sync_copy(obj_v, obj_hbm.at[wid])


def kernel(user, item, label, user_table, item_table, user_bias, item_bias, avg_rating):
    user_r = user.astype(jnp.int32).reshape(NW, NCHUNK, IDX_CHUNK)
    item_r = item.astype(jnp.int32).reshape(NW, NCHUNK, IDX_CHUNK)
    label_r = label.reshape(NW, BPW)
    ubias_f = user_bias.reshape(-1)
    ibias_f = item_bias.reshape(-1)
    avg16 = jnp.broadcast_to(jnp.asarray(avg_rating, jnp.float32).reshape(1), (16,))
    pred, loss, obj_part = _pmf_sc(user_r, item_r, label_r, user_table,
                                   item_table, ubias_f, ibias_f, avg16)
    return (pred.reshape(-1), loss.reshape(-1), jnp.sum(obj_part))
